# baseline (speedup 1.0000x reference)
import functools

import jax
import jax.numpy as jnp
from jax import lax
from jax.experimental import pallas as pl
from jax.experimental.pallas import tpu as pltpu
from jax.experimental.pallas import tpu_sc as plsc


def kernel(x, pos_table):
    B, L, D = x.shape
    NW = 32
    rows_per_w = B // NW
    n_vec = (L * D) // 16
    mesh = plsc.VectorSubcoreMesh(core_axis_name="c", subcore_axis_name="s")

    xf = x.reshape(B, L * D)
    posf = pos_table.reshape(L * D)

    @functools.partial(
        pl.kernel,
        out_type=jax.ShapeDtypeStruct((B, L * D), jnp.float32),
        mesh=mesh,
        scratch_types=[
            pltpu.VMEM((L * D,), jnp.float32),
            pltpu.VMEM((L * D,), jnp.float32),
        ],
    )
    def sc_add(x_hbm, pos_hbm, out_hbm, pos_v, buf_v):
        wid = lax.axis_index("s") * 2 + lax.axis_index("c")
        pltpu.sync_copy(pos_hbm, pos_v)

        def row_body(i, carry):
            b = wid * rows_per_w + i
            pltpu.sync_copy(x_hbm.at[b], buf_v)

            def add_body(j, c):
                o = pl.multiple_of(j * 16, 16)
                buf_v[pl.ds(o, 16)] = buf_v[pl.ds(o, 16)] + pos_v[pl.ds(o, 16)]
                return c

            lax.fori_loop(0, n_vec, add_body, 0)
            pltpu.sync_copy(buf_v, out_hbm.at[b])
            return carry

        lax.fori_loop(0, rows_per_w, row_body, 0)

    return sc_add(xf, posf).reshape(B, L, D)


# speedup vs baseline: 1.9423x; 1.9423x over previous
import functools

import jax
import jax.numpy as jnp
from jax import lax
from jax.experimental import pallas as pl
from jax.experimental.pallas import tpu as pltpu
from jax.experimental.pallas import tpu_sc as plsc


def kernel(x, pos_table):
    B, L, D = x.shape
    NW = 32
    rows_per_w = B // NW
    row_words = L * D
    mesh = plsc.VectorSubcoreMesh(core_axis_name="c", subcore_axis_name="s")

    xf = x.reshape(B, row_words)
    posf = pos_table.reshape(row_words)

    @functools.partial(
        pl.kernel,
        out_type=jax.ShapeDtypeStruct((B, row_words), jnp.float32),
        mesh=mesh,
        scratch_types=[
            pltpu.VMEM((row_words,), jnp.float32),
            pltpu.VMEM((row_words,), jnp.float32),
            pltpu.VMEM((row_words,), jnp.float32),
            pltpu.VMEM((row_words,), jnp.float32),
            pltpu.VMEM((row_words,), jnp.float32),
            pltpu.SemaphoreType.DMA,
            pltpu.SemaphoreType.DMA,
            pltpu.SemaphoreType.DMA,
            pltpu.SemaphoreType.DMA,
        ],
    )
    def sc_add(x_hbm, pos_hbm, out_hbm, pos_v, in0, in1, ob0, ob1,
               si0, si1, so0, so1):
        wid = lax.axis_index("s") * 2 + lax.axis_index("c")
        base = wid * rows_per_w
        pltpu.sync_copy(pos_hbm, pos_v)

        ibufs = (in0, in1)
        obufs = (ob0, ob1)
        isems = (si0, si1)
        osems = (so0, so1)
        h_in = [None, None]
        h_out = [None, None]

        h_in[0] = pltpu.async_copy(x_hbm.at[base + 0], in0, si0)
        h_in[1] = pltpu.async_copy(x_hbm.at[base + 1], in1, si1)

        for i in range(rows_per_w):
            p = i % 2
            src, dst = ibufs[p], obufs[p]
            h_in[p].wait()
            if h_out[p] is not None:
                h_out[p].wait()

            def add_body(j, c, src=src, dst=dst):
                o = pl.multiple_of(j * 128, 128)
                for u in range(8):
                    s = o + u * 16
                    dst[pl.ds(s, 16)] = src[pl.ds(s, 16)] + pos_v[pl.ds(s, 16)]
                return c

            lax.fori_loop(0, row_words // 128, add_body, 0)

            h_out[p] = pltpu.async_copy(dst, out_hbm.at[base + i], osems[p])
            if i + 2 < rows_per_w:
                h_in[p] = pltpu.async_copy(
                    x_hbm.at[base + i + 2], ibufs[p], isems[p])

        h_out[0].wait()
        h_out[1].wait()

    return sc_add(xf, posf).reshape(B, L, D)


# speedup vs baseline: 4.8099x; 2.4764x over previous
import functools

import jax
import jax.numpy as jnp
from jax import lax
from jax.experimental import pallas as pl
from jax.experimental.pallas import tpu as pltpu
from jax.experimental.pallas import tpu_sc as plsc


def kernel(x, pos_table):
    B, L, D = x.shape
    NW = 32
    rows_per_w = B // NW
    mesh = plsc.VectorSubcoreMesh(core_axis_name="c", subcore_axis_name="s")

    @functools.partial(
        pl.kernel,
        out_type=jax.ShapeDtypeStruct((B, L, D), jnp.float32),
        mesh=mesh,
        scratch_types=[
            pltpu.VMEM((L, D), jnp.float32),
            pltpu.VMEM((L, D), jnp.float32),
            pltpu.VMEM((L, D), jnp.float32),
            pltpu.VMEM((L, D), jnp.float32),
            pltpu.VMEM((L, D), jnp.float32),
            pltpu.SemaphoreType.DMA,
            pltpu.SemaphoreType.DMA,
            pltpu.SemaphoreType.DMA,
            pltpu.SemaphoreType.DMA,
        ],
    )
    def sc_add(x_hbm, pos_hbm, out_hbm, pos_v, in0, in1, ob0, ob1,
               si0, si1, so0, so1):
        wid = lax.axis_index("s") * 2 + lax.axis_index("c")
        base = wid * rows_per_w
        pltpu.sync_copy(pos_hbm, pos_v)

        ibufs = (in0, in1)
        obufs = (ob0, ob1)
        isems = (si0, si1)
        osems = (so0, so1)
        h_in = [None, None]
        h_out = [None, None]

        h_in[0] = pltpu.async_copy(x_hbm.at[base + 0], in0, si0)
        h_in[1] = pltpu.async_copy(x_hbm.at[base + 1], in1, si1)

        for i in range(rows_per_w):
            p = i % 2
            src, dst = ibufs[p], obufs[p]
            h_in[p].wait()
            if h_out[p] is not None:
                h_out[p].wait()

            def add_body(r, c, src=src, dst=dst):
                for u in range(8):
                    sl = pl.ds(u * 16, 16)
                    dst[r, sl] = src[r, sl] + pos_v[r, sl]
                return c

            lax.fori_loop(0, L, add_body, 0)

            h_out[p] = pltpu.async_copy(dst, out_hbm.at[base + i], osems[p])
            if i + 2 < rows_per_w:
                h_in[p] = pltpu.async_copy(
                    x_hbm.at[base + i + 2], ibufs[p], isems[p])

        h_out[0].wait()
        h_out[1].wait()

    return sc_add(x, pos_table)


# speedup vs baseline: 4.8494x; 1.0082x over previous
import functools

import jax
import jax.numpy as jnp
from jax import lax
from jax.experimental import pallas as pl
from jax.experimental.pallas import tpu as pltpu
from jax.experimental.pallas import tpu_sc as plsc


def kernel(x, pos_table):
    B, L, D = x.shape
    NW = 32
    rows_per_w = B // NW
    NBUF = 4
    mesh = plsc.VectorSubcoreMesh(core_axis_name="c", subcore_axis_name="s")

    @functools.partial(
        pl.kernel,
        out_type=jax.ShapeDtypeStruct((B, L, D), jnp.float32),
        mesh=mesh,
        scratch_types=[
            pltpu.VMEM((L, D), jnp.float32),
            pltpu.VMEM((L, D), jnp.float32),
            pltpu.VMEM((L, D), jnp.float32),
            pltpu.VMEM((L, D), jnp.float32),
            pltpu.VMEM((L, D), jnp.float32),
            pltpu.SemaphoreType.DMA,
            pltpu.SemaphoreType.DMA,
            pltpu.SemaphoreType.DMA,
            pltpu.SemaphoreType.DMA,
            pltpu.SemaphoreType.DMA,
            pltpu.SemaphoreType.DMA,
            pltpu.SemaphoreType.DMA,
            pltpu.SemaphoreType.DMA,
        ],
    )
    def sc_add(x_hbm, pos_hbm, out_hbm, pos_v, b0, b1, b2, b3,
               si0, si1, si2, si3, so0, so1, so2, so3):
        wid = lax.axis_index("s") * 2 + lax.axis_index("c")
        base = wid * rows_per_w
        pltpu.sync_copy(pos_hbm, pos_v)

        bufs = (b0, b1, b2, b3)
        isems = (si0, si1, si2, si3)
        osems = (so0, so1, so2, so3)
        h_in = [None] * NBUF
        h_out = [None] * NBUF

        h_in[0] = pltpu.async_copy(x_hbm.at[base + 0], bufs[0], isems[0])
        h_in[1] = pltpu.async_copy(x_hbm.at[base + 1], bufs[1], isems[1])

        for i in range(rows_per_w):
            p = i % NBUF
            nxt = i + 2
            if nxt < rows_per_w:
                q = nxt % NBUF
                if h_out[q] is not None:
                    h_out[q].wait()
                h_in[q] = pltpu.async_copy(x_hbm.at[base + nxt], bufs[q],
                                           isems[q])

            buf = bufs[p]
            h_in[p].wait()

            def add_body(r, c, buf=buf):
                for u in range(8):
                    sl = pl.ds(u * 16, 16)
                    plsc.addupdate(buf.at[r, sl], pos_v[r, sl])
                return c

            lax.fori_loop(0, L, add_body, 0)

            h_out[p] = pltpu.async_copy(buf, out_hbm.at[base + i], osems[p])

        for p in range(NBUF):
            h_out[p].wait()

    return sc_add(x, pos_table)


# speedup vs baseline: 7.7928x; 1.6070x over previous
import jax
import jax.numpy as jnp
from jax.experimental import pallas as pl
from jax.experimental.pallas import tpu as pltpu


def _add_kernel(x_ref, pos_ref, out_ref):
    out_ref[...] = x_ref[...] + pos_ref[...][None]


def kernel(x, pos_table):
    B, L, D = x.shape
    BLK = 128
    grid = (B // BLK,)
    return pl.pallas_call(
        _add_kernel,
        grid=grid,
        in_specs=[
            pl.BlockSpec((BLK, L, D), lambda i: (i, 0, 0)),
            pl.BlockSpec((L, D), lambda i: (0, 0)),
        ],
        out_specs=pl.BlockSpec((BLK, L, D), lambda i: (i, 0, 0)),
        out_shape=jax.ShapeDtypeStruct((B, L, D), x.dtype),
    )(x, pos_table)
